# baseline (speedup 1.0000x reference)
import functools

import jax
import jax.numpy as jnp
from jax import lax
from jax.experimental import pallas as pl
from jax.experimental.pallas import tpu as pltpu
from jax.experimental.pallas import tpu_sc as plsc

_VOCAB = 1000000
_D = 64
_MAX_LEN = 200
_BATCH = 4096
_B = _BATCH * _MAX_LEN

_NC = 2
_NS = 16
_NW = _NC * _NS
_PER_W = _B // _NW
_C = 200
_NCHUNK = _PER_W // _C
_LANES = 16


def _positional() -> jax.Array:
    position = jnp.arange(0, _MAX_LEN, dtype=jnp.float32).reshape(-1, 1)
    div = jnp.exp(
        jnp.arange(0, _D, 2, dtype=jnp.float32) / _D
        * -jnp.log(jnp.float32(10000.0))
    )
    p = jnp.zeros((_MAX_LEN, _D), dtype=jnp.float32)
    p = p.at[:, 0::2].set(jnp.sin(position * div))
    p = p.at[:, 1::2].set(jnp.cos(position * div))
    return p


_mesh = plsc.VectorSubcoreMesh(core_axis_name="c", subcore_axis_name="s")


@functools.partial(
    pl.kernel,
    mesh=_mesh,
    out_type=jax.ShapeDtypeStruct((_B, _D), jnp.float32),
    scratch_types=[
        pltpu.VMEM((_C,), jnp.int32),
        pltpu.VMEM((_C, _D), jnp.float32),
        pltpu.VMEM((_MAX_LEN, _D), jnp.float32),
        pltpu.SemaphoreType.DMA,
    ],
    compiler_params=pltpu.CompilerParams(use_tc_tiling_on_sc=False),
)
def _embed(x_hbm, table_hbm, p_hbm, out_hbm, idx_v, rows_v, p_v, sem):
    wid = lax.axis_index("s") * _NC + lax.axis_index("c")
    base = wid * _PER_W
    pltpu.sync_copy(p_hbm, p_v)

    def chunk_body(k, carry):
        off = base + k * _C
        pltpu.sync_copy(x_hbm.at[pl.ds(off, _C)], idx_v)
        pltpu.async_copy(table_hbm.at[idx_v], rows_v, sem).wait()

        def row_body(r, c2):
            for d in range(_D // _LANES):
                sl = pl.ds(d * _LANES, _LANES)
                rows_v[r, sl] = rows_v[r, sl] + p_v[r, sl]
            return c2

        lax.fori_loop(0, _C, row_body, 0)
        pltpu.sync_copy(rows_v, out_hbm.at[pl.ds(off, _C)])
        return carry

    lax.fori_loop(0, _NCHUNK, chunk_body, 0)


def kernel(X, table):
    p = _positional()
    xf = X.reshape(-1)
    out = _embed(xf, table, p)
    return out.reshape(_BATCH, _MAX_LEN, _D)
